# baseline (speedup 1.0000x reference)
import functools

import jax
import jax.numpy as jnp
from jax.experimental import pallas as pl

NN_NUM = 8
RADIUS_QUERY = 0.08

Q = 1024
KPTS = 100000
BK = 2048
NB = 49
KPAD = NB * BK


def _d2_body(p_ref, ct_ref, out_ref):
    p = p_ref[...]
    ct = ct_ref[...]
    dotv = jax.lax.dot_general(p, ct, (((1,), (0,)), ((), ())),
                               preferred_element_type=jnp.float32)
    x = p[:, 0:1]
    y = p[:, 1:2]
    z = p[:, 2:3]
    qsq = x * x + y * y + z * z
    cx = ct[0:1, :]
    cy = ct[1:2, :]
    cz = ct[2:3, :]
    ksq = cx * cx + cy * cy + cz * cz
    out_ref[...] = (qsq - 2.0 * dotv) + ksq


def kernel(pos, cloud_pos):
    ct = jnp.concatenate(
        [cloud_pos.T, jnp.full((3, KPAD - KPTS), 1e4, jnp.float32)], axis=1)

    d2 = pl.pallas_call(
        _d2_body,
        grid=(NB,),
        in_specs=[
            pl.BlockSpec((Q, 3), lambda i: (0, 0)),
            pl.BlockSpec((3, BK), lambda i: (0, i)),
        ],
        out_specs=pl.BlockSpec((Q, BK), lambda i: (0, i)),
        out_shape=jax.ShapeDtypeStruct((Q, KPAD), jnp.float32),
    )(pos, ct)

    d2 = d2[:, :KPTS]
    neg_top, I = jax.lax.top_k(-d2, NN_NUM)
    D = -neg_top
    neighbor_num = jnp.sum(D < RADIUS_QUERY ** 2, axis=-1).astype(jnp.int32)
    return D, I, neighbor_num


# speedup vs baseline: 8.3806x; 8.3806x over previous
import functools

import jax
import jax.numpy as jnp
from jax import lax
from jax.experimental import pallas as pl
from jax.experimental.pallas import tpu as pltpu
from jax.experimental.pallas import tpu_sc as plsc

NN_NUM = 8
RADIUS_QUERY = 0.08

Q = 1024
KPTS = 100000
BK = 2048
NB = 49
KPAD = NB * BK
NSEL = 10
TMAX = 64
CAND = NSEL * TMAX
INF = 3e38
BIGI = 2**30


def _d2_body(p_ref, ct_ref, d2_ref, acc_ref):
    i = pl.program_id(0)
    p = p_ref[...]
    ct = ct_ref[...]
    dotv = lax.dot_general(p, ct, (((1,), (0,)), ((), ())),
                           preferred_element_type=jnp.float32)
    x = p[:, 0:1]
    y = p[:, 1:2]
    z = p[:, 2:3]
    qsq = (x * x + z * z) + y * y
    cx = ct[0:1, :]
    cy = ct[1:2, :]
    cz = ct[2:3, :]
    ksq = (cx * cx + cz * cz) + cy * cy
    d2 = (qsq - 2.0 * dotv) + ksq
    for j in range(BK // 128):
        d2_ref[0, j] = d2[:, 128 * j:128 * (j + 1)]

    @pl.when(i == 0)
    def _():
        acc_ref[...] = d2

    @pl.when(i > 0)
    def _():
        acc_ref[...] = jnp.minimum(acc_ref[...], d2)


def _select_idx_body(acc_ref, ipt_ref, iflat_ref):
    work = acc_ref[...]
    iota = lax.broadcasted_iota(jnp.int32, (Q, BK), 1)
    lanes = []
    for _ in range(NSEL):
        m = jnp.min(work, axis=1, keepdims=True)
        mi = jnp.where(work == m, iota, BIGI)
        lane = jnp.min(mi, axis=1, keepdims=True)
        lanes.append(lane)
        work = jnp.where(iota == lane, INF, work)

    t64 = lax.broadcasted_iota(jnp.int32, (Q, TMAX), 1)
    cols = []
    for s in range(NSEL):
        idx_s = jnp.where(t64 < NB, lanes[s] + BK * t64, lanes[s])
        cols.append(idx_s)
    ipt = jnp.concatenate(cols, axis=1)
    ipt_ref[...] = ipt
    qrow = lax.broadcasted_iota(jnp.int32, (Q, CAND), 0) * 128
    iflat_ref[...] = (ipt >> 7) * (Q * 128) + qrow + (ipt & 127)


def _final_body(cand_ref, ipt_ref, d_ref, i_ref, nn_ref):
    work = cand_ref[...]
    ip = ipt_ref[...]
    tpos = lax.broadcasted_iota(jnp.int32, (Q, CAND), 1) % TMAX
    work = jnp.where(tpos < NB, work, INF)
    ds, is_ = [], []
    for _ in range(NN_NUM):
        m = jnp.min(work, axis=1, keepdims=True)
        wi = jnp.where(work == m, ip, BIGI)
        pick = jnp.min(wi, axis=1, keepdims=True)
        ds.append(m)
        is_.append(pick)
        work = jnp.where(wi == pick, INF, work)
    D = jnp.concatenate(ds, axis=1)
    I = jnp.concatenate(is_, axis=1)
    d_ref[...] = D
    i_ref[...] = I
    nn_ref[...] = jnp.sum(
        (D < RADIUS_QUERY ** 2).astype(jnp.int32), axis=1, keepdims=True)


NW = 32
QPW = Q // NW
NCH = CAND // 128


def _make_gather_kernel():
    mesh = plsc.VectorSubcoreMesh(core_axis_name="c", subcore_axis_name="s")

    @functools.partial(
        pl.kernel,
        mesh=mesh,
        out_type=jax.ShapeDtypeStruct((Q, NCH, 128), jnp.float32),
        scratch_types=[
            pltpu.VMEM((NCH, 128), jnp.int32),
            pltpu.VMEM((NCH, 128), jnp.float32),
            pltpu.SemaphoreType.DMA,
        ],
    )
    def gather_k(d2flat_hbm, idx_hbm, out_hbm, idx_v, val_v, sem):
        cid = lax.axis_index("c")
        sid = lax.axis_index("s")
        wid = sid * 2 + cid

        def body(j, carry):
            q = wid * QPW + j
            pltpu.sync_copy(idx_hbm.at[q], idx_v)
            cps = [
                pltpu.async_copy(
                    d2flat_hbm.at[idx_v.at[c]],
                    val_v.at[c],
                    sem,
                )
                for c in range(NCH)
            ]
            for cp in cps:
                cp.wait()
            pltpu.sync_copy(val_v, out_hbm.at[q])
            return carry

        lax.fori_loop(0, QPW, body, 0)

    return gather_k


def kernel(pos, cloud_pos):
    ct = jnp.concatenate(
        [cloud_pos.T, jnp.full((3, KPAD - KPTS), 1e4, jnp.float32)], axis=1)

    d2, acc = pl.pallas_call(
        _d2_body,
        grid=(NB,),
        in_specs=[
            pl.BlockSpec((Q, 3), lambda i: (0, 0)),
            pl.BlockSpec((3, BK), lambda i: (0, i)),
        ],
        out_specs=[
            pl.BlockSpec((1, BK // 128, Q, 128), lambda i: (i, 0, 0, 0)),
            pl.BlockSpec((Q, BK), lambda i: (0, 0)),
        ],
        out_shape=[
            jax.ShapeDtypeStruct((NB, BK // 128, Q, 128), jnp.float32),
            jax.ShapeDtypeStruct((Q, BK), jnp.float32),
        ],
    )(pos, ct)

    ipt, iflat = pl.pallas_call(
        _select_idx_body,
        out_shape=[
            jax.ShapeDtypeStruct((Q, CAND), jnp.int32),
            jax.ShapeDtypeStruct((Q, CAND), jnp.int32),
        ],
    )(acc)

    gather_k = _make_gather_kernel()
    cand = gather_k(d2.reshape(Q * KPAD), iflat.reshape(Q, NCH, 128))
    cand = cand.reshape(Q, CAND)

    D, I, nn = pl.pallas_call(
        _final_body,
        out_shape=[
            jax.ShapeDtypeStruct((Q, NN_NUM), jnp.float32),
            jax.ShapeDtypeStruct((Q, NN_NUM), jnp.int32),
            jax.ShapeDtypeStruct((Q, 1), jnp.int32),
        ],
    )(cand, ipt)

    return D, I, nn.reshape(Q)


# speedup vs baseline: 9.8311x; 1.1731x over previous
import functools

import jax
import jax.numpy as jnp
from jax import lax
from jax.experimental import pallas as pl
from jax.experimental.pallas import tpu as pltpu
from jax.experimental.pallas import tpu_sc as plsc

NN_NUM = 8
RADIUS_QUERY = 0.08

Q = 1024
KPTS = 100000
BK = 2048
NB = 49
KPAD = NB * BK
NSEL = 10
CAND = 512
INF = 3e38
BIGI = 2**30

NW = 32
QPW = Q // NW
NCH = CAND // 128


def _d2_body(p_ref, ct_ref, d2_ref, ipt_ref, iflat_ref, acc_ref):
    i = pl.program_id(0)
    p = p_ref[...]
    ct = ct_ref[...]
    dotv = lax.dot_general(p, ct, (((1,), (0,)), ((), ())),
                           preferred_element_type=jnp.float32)
    x = p[:, 0:1]
    y = p[:, 1:2]
    z = p[:, 2:3]
    qsq = (x * x + z * z) + y * y
    cx = ct[0:1, :]
    cy = ct[1:2, :]
    cz = ct[2:3, :]
    ksq = (cx * cx + cz * cz) + cy * cy
    d2 = (qsq - 2.0 * dotv) + ksq
    for j in range(BK // 128):
        d2_ref[0, j] = d2[:, 128 * j:128 * (j + 1)]

    @pl.when(i == 0)
    def _():
        acc_ref[...] = d2

    @pl.when(i > 0)
    def _():
        acc_ref[...] = jnp.minimum(acc_ref[...], d2)

    @pl.when(i == NB - 1)
    def _():
        work = acc_ref[...]
        iota = lax.broadcasted_iota(jnp.int32, (Q, BK), 1)
        lanes = []
        for _ in range(NSEL):
            m = jnp.min(work, axis=1, keepdims=True)
            mi = jnp.where(work == m, iota, BIGI)
            lane = jnp.min(mi, axis=1, keepdims=True)
            lanes.append(lane)
            work = jnp.where(iota == lane, INF, work)

        t49 = lax.broadcasted_iota(jnp.int32, (Q, NB), 1)
        cols = [lanes[s] + BK * t49 for s in range(NSEL)]
        cols.append(jnp.zeros((Q, CAND - NSEL * NB), jnp.int32))
        ipt = jnp.concatenate(cols, axis=1)
        ipt_ref[...] = ipt
        qrow = lax.broadcasted_iota(jnp.int32, (Q, CAND), 0) * 128
        iflat_ref[...] = (ipt >> 7) * (Q * 128) + qrow + (ipt & 127)


def _final_body(cand_ref, ipt_ref, d_ref, i_ref, nn_ref):
    work = cand_ref[...]
    ip = ipt_ref[...]
    cpos = lax.broadcasted_iota(jnp.int32, (Q, CAND), 1)
    work = jnp.where(cpos < NSEL * NB, work, INF)
    ds, is_ = [], []
    for _ in range(NN_NUM):
        m = jnp.min(work, axis=1, keepdims=True)
        wi = jnp.where(work == m, ip, BIGI)
        pick = jnp.min(wi, axis=1, keepdims=True)
        ds.append(m)
        is_.append(pick)
        work = jnp.where(wi == pick, INF, work)
    D = jnp.concatenate(ds, axis=1)
    I = jnp.concatenate(is_, axis=1)
    d_ref[...] = D
    i_ref[...] = I
    nn_ref[...] = jnp.sum(
        (D < RADIUS_QUERY ** 2).astype(jnp.int32), axis=1, keepdims=True)


def _make_gather_kernel():
    mesh = plsc.VectorSubcoreMesh(core_axis_name="c", subcore_axis_name="s")

    @functools.partial(
        pl.kernel,
        mesh=mesh,
        out_type=jax.ShapeDtypeStruct((Q, NCH, 128), jnp.float32),
        scratch_types=[
            pltpu.VMEM((2, NCH, 128), jnp.int32),
            pltpu.VMEM((2, NCH, 128), jnp.float32),
            pltpu.SemaphoreType.DMA,
            pltpu.SemaphoreType.DMA,
        ],
    )
    def gather_k(d2flat_hbm, idx_hbm, out_hbm, idx_v, val_v, sem0, sem1):
        cid = lax.axis_index("c")
        sid = lax.axis_index("s")
        wid = sid * 2 + cid
        q0 = wid * QPW
        sems = [sem0, sem1]

        def fire(j, b):
            q = q0 + j
            pltpu.sync_copy(idx_hbm.at[q], idx_v.at[b])
            return [
                pltpu.async_copy(
                    d2flat_hbm.at[idx_v.at[b, c]],
                    val_v.at[b, c],
                    sems[b],
                )
                for c in range(NCH)
            ]

        cps = fire(0, 0)
        for j in range(QPW):
            b = j % 2
            if j + 1 < QPW:
                nxt = fire(j + 1, 1 - b)
            for cp in cps:
                cp.wait()
            pltpu.sync_copy(val_v.at[b], out_hbm.at[q0 + j])
            if j + 1 < QPW:
                cps = nxt

    return gather_k


def kernel(pos, cloud_pos):
    ct = jnp.concatenate(
        [cloud_pos.T, jnp.full((3, KPAD - KPTS), 1e4, jnp.float32)], axis=1)

    d2, ipt, iflat = pl.pallas_call(
        _d2_body,
        grid=(NB,),
        in_specs=[
            pl.BlockSpec((Q, 3), lambda i: (0, 0)),
            pl.BlockSpec((3, BK), lambda i: (0, i)),
        ],
        out_specs=[
            pl.BlockSpec((1, BK // 128, Q, 128), lambda i: (i, 0, 0, 0)),
            pl.BlockSpec((Q, CAND), lambda i: (0, 0)),
            pl.BlockSpec((Q, CAND), lambda i: (0, 0)),
        ],
        out_shape=[
            jax.ShapeDtypeStruct((NB, BK // 128, Q, 128), jnp.float32),
            jax.ShapeDtypeStruct((Q, CAND), jnp.int32),
            jax.ShapeDtypeStruct((Q, CAND), jnp.int32),
        ],
        scratch_shapes=[pltpu.VMEM((Q, BK), jnp.float32)],
    )(pos, ct)

    gather_k = _make_gather_kernel()
    cand = gather_k(d2.reshape(NB * (BK // 128) * Q * 128),
                    iflat.reshape(Q, NCH, 128))
    cand = cand.reshape(Q, CAND)

    D, I, nn = pl.pallas_call(
        _final_body,
        out_shape=[
            jax.ShapeDtypeStruct((Q, NN_NUM), jnp.float32),
            jax.ShapeDtypeStruct((Q, NN_NUM), jnp.int32),
            jax.ShapeDtypeStruct((Q, 1), jnp.int32),
        ],
    )(cand, ipt)

    return D, I, nn.reshape(Q)
